# Initial kernel scaffold; baseline (speedup 1.0000x reference)
import jax
import jax.numpy as jnp
from jax.experimental import pallas as pl


def kernel(indices, table):
    raise NotImplementedError("write your pallas kernel here")


# speedup vs baseline: 2.9769x; 2.9769x over previous
import functools

import jax
import jax.numpy as jnp
from jax import lax
from jax.experimental import pallas as pl
from jax.experimental.pallas import tpu as pltpu
from jax.experimental.pallas import tpu_sc as plsc

D = 128
NC, NS = 2, 16
NW = NC * NS
CHUNK = 128


@functools.partial(jax.jit, static_argnames=("n_chunks",))
def _sc_gather(idx3, table, n_chunks):
    b_per_w = n_chunks * CHUNK
    B = NW * b_per_w
    mesh = plsc.VectorSubcoreMesh(core_axis_name="c", subcore_axis_name="s")

    @functools.partial(
        pl.kernel,
        mesh=mesh,
        out_type=jax.ShapeDtypeStruct((B, D), jnp.float32),
        scratch_types=[
            pltpu.VMEM((n_chunks, CHUNK), jnp.int32),
            pltpu.VMEM((CHUNK, D), jnp.float32),
            pltpu.SemaphoreType.DMA,
        ],
    )
    def k(table_hbm, idx_hbm, out_hbm, idx_v, rows_v, gsem):
        wid = lax.axis_index("s") * NC + lax.axis_index("c")
        base = wid * b_per_w
        pltpu.sync_copy(idx_hbm.at[wid], idx_v)

        def body(j, carry):
            pltpu.async_copy(table_hbm.at[idx_v.at[j]], rows_v, gsem).wait()
            pltpu.sync_copy(rows_v, out_hbm.at[pl.ds(base + j * CHUNK, CHUNK)])
            return carry

        lax.fori_loop(0, n_chunks, body, 0)

    return k(table, idx3)


def kernel(indices, table):
    batch, hist = indices.shape
    B = batch * hist
    n_chunks = B // (NW * CHUNK)
    idx3 = indices.reshape(NW, n_chunks, CHUNK).astype(jnp.int32)
    out = _sc_gather(idx3, table, n_chunks)
    return out.reshape(batch, hist, D)


# speedup vs baseline: 3.3385x; 1.1215x over previous
import functools

import jax
import jax.numpy as jnp
from jax import lax
from jax.experimental import pallas as pl
from jax.experimental.pallas import tpu as pltpu
from jax.experimental.pallas import tpu_sc as plsc

D = 128
NC, NS = 2, 16
NW = NC * NS
CHUNK = 128
NBUF = 2


@functools.partial(jax.jit, static_argnames=("n_chunks",))
def _sc_gather(idx3, table, n_chunks):
    b_per_w = n_chunks * CHUNK
    B = NW * b_per_w
    ngroups = n_chunks // NBUF
    assert n_chunks == ngroups * NBUF and ngroups >= 2
    mesh = plsc.VectorSubcoreMesh(core_axis_name="c", subcore_axis_name="s")

    @functools.partial(
        pl.kernel,
        mesh=mesh,
        out_type=jax.ShapeDtypeStruct((B, D), jnp.float32),
        scratch_types=[
            pltpu.VMEM((n_chunks, CHUNK), jnp.int32),
            pltpu.VMEM((CHUNK, D), jnp.float32),
            pltpu.VMEM((CHUNK, D), jnp.float32),
            pltpu.SemaphoreType.DMA,
            pltpu.SemaphoreType.DMA,
        ],
    )
    def k(table_hbm, idx_hbm, out_hbm, idx_v, buf0, buf1, gsem, osem):
        bufs = (buf0, buf1)
        wid = lax.axis_index("s") * NC + lax.axis_index("c")
        base = wid * b_per_w
        pltpu.sync_copy(idx_hbm.at[wid], idx_v)

        def g_copy(j, b):
            return pltpu.make_async_copy(table_hbm.at[idx_v.at[j]], bufs[b], gsem)

        def s_copy(j, b):
            return pltpu.make_async_copy(
                bufs[b], out_hbm.at[pl.ds(base + j * CHUNK, CHUNK)], osem)

        def steady(j, b):
            s_copy(j - 1, 1 - b).wait()
            g_copy(j + 1, 1 - b).start()
            g_copy(j, b).wait()
            s_copy(j, b).start()

        g_copy(0, 0).start()
        g_copy(1, 1).start()
        g_copy(0, 0).wait()
        s_copy(0, 0).start()
        steady(1, 1)

        def body(g, carry):
            j = g * NBUF
            steady(j, 0)
            steady(j + 1, 1)
            return carry

        lax.fori_loop(1, ngroups - 1, body, 0)

        jl = n_chunks - 2
        steady(jl, 0)
        s_copy(jl, 0).wait()
        g_copy(jl + 1, 1).wait()
        s_copy(jl + 1, 1).start()
        s_copy(jl + 1, 1).wait()

    return k(table, idx3)


def kernel(indices, table):
    batch, hist = indices.shape
    B = batch * hist
    n_chunks = B // (NW * CHUNK)
    idx3 = indices.reshape(NW, n_chunks, CHUNK).astype(jnp.int32)
    out = _sc_gather(idx3, table, n_chunks)
    return out.reshape(batch, hist, D)


# speedup vs baseline: 3.3652x; 1.0080x over previous
import functools

import jax
import jax.numpy as jnp
from jax import lax
from jax.experimental import pallas as pl
from jax.experimental.pallas import tpu as pltpu
from jax.experimental.pallas import tpu_sc as plsc

D = 128
NC, NS = 2, 16
NW = NC * NS
CHUNK = 128
NBUF = 5


@functools.partial(jax.jit, static_argnames=("n_chunks",))
def _sc_gather(idx3, table, n_chunks):
    b_per_w = n_chunks * CHUNK
    B = NW * b_per_w
    ngroups = n_chunks // NBUF
    assert n_chunks == ngroups * NBUF and ngroups >= 2
    mesh = plsc.VectorSubcoreMesh(core_axis_name="c", subcore_axis_name="s")

    @functools.partial(
        pl.kernel,
        mesh=mesh,
        out_type=jax.ShapeDtypeStruct((B, D), jnp.float32),
        scratch_types=[
            pltpu.VMEM((n_chunks, CHUNK), jnp.int32),
            *[pltpu.VMEM((CHUNK, D), jnp.float32) for _ in range(NBUF)],
            pltpu.SemaphoreType.DMA,
            pltpu.SemaphoreType.DMA,
        ],
    )
    def k(table_hbm, idx_hbm, out_hbm, idx_v, *rest):
        bufs = rest[:NBUF]
        gsem, osem = rest[NBUF], rest[NBUF + 1]
        wid = lax.axis_index("s") * NC + lax.axis_index("c")
        base = wid * b_per_w
        pltpu.sync_copy(idx_hbm.at[wid], idx_v)

        def g_copy(j, b):
            return pltpu.make_async_copy(table_hbm.at[idx_v.at[j]], bufs[b], gsem)

        def s_copy(j, b):
            return pltpu.make_async_copy(
                bufs[b], out_hbm.at[pl.ds(base + j * CHUNK, CHUNK)], osem)

        def steady(j, b):
            prev = (b - 1) % NBUF
            s_copy(j - 1, prev).wait()
            g_copy(j + NBUF - 1, prev).start()
            g_copy(j, b).wait()
            s_copy(j, b).start()

        def tail(j, b):
            s_copy(j - 1, (b - 1) % NBUF).wait()
            g_copy(j, b).wait()
            s_copy(j, b).start()

        for b in range(NBUF):
            g_copy(b, b).start()
        g_copy(0, 0).wait()
        s_copy(0, 0).start()
        for b in range(1, NBUF):
            steady(b, b)

        def body(g, carry):
            j = g * NBUF
            for b in range(NBUF):
                steady(j + b, b)
            return carry

        lax.fori_loop(1, ngroups - 1, body, 0)

        jl = n_chunks - NBUF
        steady(jl, 0)
        for b in range(1, NBUF):
            tail(jl + b, b)
        s_copy(n_chunks - 1, NBUF - 1).wait()

    return k(table, idx3)


def kernel(indices, table):
    batch, hist = indices.shape
    B = batch * hist
    n_chunks = B // (NW * CHUNK)
    idx3 = indices.reshape(NW, n_chunks, CHUNK).astype(jnp.int32)
    out = _sc_gather(idx3, table, n_chunks)
    return out.reshape(batch, hist, D)


# speedup vs baseline: 5.9944x; 1.7813x over previous
import functools

import jax
import jax.numpy as jnp
from jax import lax
from jax.experimental import pallas as pl
from jax.experimental.pallas import tpu as pltpu
from jax.experimental.pallas import tpu_sc as plsc

D = 128
NC, NS = 2, 16
NW = NC * NS
BPC = 2
NBUF = 4


@functools.partial(jax.jit, static_argnames=("batch", "hist"))
def _sc_gather(idx3, table, batch, hist):
    e_per_w = batch // NW
    n_chunks = e_per_w // BPC
    rows_per_chunk = BPC * hist
    ngroups = n_chunks // NBUF
    assert n_chunks == ngroups * NBUF and ngroups >= 2
    mesh = plsc.VectorSubcoreMesh(core_axis_name="c", subcore_axis_name="s")

    @functools.partial(
        pl.kernel,
        mesh=mesh,
        out_type=jax.ShapeDtypeStruct((batch, hist, D), jnp.float32),
        scratch_types=[
            pltpu.VMEM((n_chunks, rows_per_chunk), jnp.int32),
            *[pltpu.VMEM((rows_per_chunk, D), jnp.float32) for _ in range(NBUF)],
            pltpu.SemaphoreType.DMA,
            pltpu.SemaphoreType.DMA,
        ],
    )
    def k(table_hbm, idx_hbm, out_hbm, idx_v, *rest):
        bufs = rest[:NBUF]
        gsem, osem = rest[NBUF], rest[NBUF + 1]
        wid = lax.axis_index("s") * NC + lax.axis_index("c")
        base = wid * e_per_w
        pltpu.sync_copy(idx_hbm.at[wid], idx_v)

        def g_copy(j, b):
            return pltpu.make_async_copy(table_hbm.at[idx_v.at[j]], bufs[b], gsem)

        def s_copy(j, b, t):
            return pltpu.make_async_copy(
                bufs[b].at[pl.ds(t * hist, hist)],
                out_hbm.at[base + j * BPC + t], osem)

        def start_s(j, b):
            for t in range(BPC):
                s_copy(j, b, t).start()

        def wait_s(j, b):
            for t in range(BPC):
                s_copy(j, b, t).wait()

        def steady(j, b):
            prev = (b - 1) % NBUF
            wait_s(j - 1, prev)
            g_copy(j + NBUF - 1, prev).start()
            g_copy(j, b).wait()
            start_s(j, b)

        def tail(j, b):
            wait_s(j - 1, (b - 1) % NBUF)
            g_copy(j, b).wait()
            start_s(j, b)

        for b in range(NBUF):
            g_copy(b, b).start()
        g_copy(0, 0).wait()
        start_s(0, 0)
        for b in range(1, NBUF):
            steady(b, b)

        def body(g, carry):
            j = g * NBUF
            for b in range(NBUF):
                steady(j + b, b)
            return carry

        lax.fori_loop(1, ngroups - 1, body, 0)

        jl = n_chunks - NBUF
        steady(jl, 0)
        for b in range(1, NBUF):
            tail(jl + b, b)
        wait_s(n_chunks - 1, NBUF - 1)

    return k(table, idx3)


def kernel(indices, table):
    batch, hist = indices.shape
    n_chunks = batch // (NW * BPC)
    idx3 = indices.reshape(NW, n_chunks, BPC * hist).astype(jnp.int32)
    return _sc_gather(idx3, table, batch, hist)
